# Initial kernel scaffold; baseline (speedup 1.0000x reference)
import jax
import jax.numpy as jnp
from jax import lax
from jax.experimental import pallas as pl
from jax.experimental.pallas import tpu as pltpu
from jax.experimental.pallas import tpu_sc as plsc

_N, _D, _R, _T = 100000, 128, 500, 200000
_REG = 0.01
_NC, _NS = 2, 16
_NW = _NC * _NS
_CB = 128
_NCHUNK = 50
_TPW = _CB * _NCHUNK
_TP = _NW * _TPW
_PAD = _TP - _T

_GB = 10
_EB = _N // _GB
_SROWS = _TP // _D
_SB = _SROWS // _GB
_WPAD = 512


def _score_body(embed, hidx_h, ridx_h, tidx_h, wrel, out,
                hidx, ridx, tidx, sbuf, rbuf, obuf, scbuf,
                gsem0, gsem1, ssem0, ssem1):
    wid = lax.axis_index("s") * _NC + lax.axis_index("c")
    base = wid * _TPW
    pltpu.sync_copy(hidx_h.at[pl.ds(base, _TPW)], hidx)
    pltpu.sync_copy(ridx_h.at[pl.ds(base, _TPW)], ridx)
    pltpu.sync_copy(tidx_h.at[pl.ds(base, _TPW)], tidx)

    gsems = (gsem0, gsem1)
    ssems = (ssem0, ssem1)

    def gather_descs(c, b):
        off = c * _CB
        return (
            pltpu.make_async_copy(embed.at[hidx.at[pl.ds(off, _CB)]],
                                  sbuf.at[b], gsems[b]),
            pltpu.make_async_copy(wrel.at[ridx.at[pl.ds(off, _CB)]],
                                  rbuf.at[b], gsems[b]),
            pltpu.make_async_copy(embed.at[tidx.at[pl.ds(off, _CB)]],
                                  obuf.at[b], gsems[b]),
        )

    def fire(c, b):
        for dsc in gather_descs(c, b):
            dsc.start()

    def wait_gathers(c, b):
        for dsc in gather_descs(c, b):
            dsc.wait()

    def compute(c, b):
        for g in range(_CB // 16):
            rows = lax.iota(jnp.int32, 16) + (g * 16)

            def dstep(d, acc):
                dv = jnp.full((16,), d, jnp.int32)
                sv = plsc.load_gather(sbuf.at[b], [rows, dv])
                rv = plsc.load_gather(rbuf.at[b], [rows, dv])
                ov = plsc.load_gather(obuf.at[b], [rows, dv])
                return acc + sv * rv * ov

            acc = lax.fori_loop(0, _D, dstep, jnp.zeros((16,), jnp.float32),
                                unroll=4)
            scbuf[b, pl.ds(g * 16, 16)] = acc

    fire(0, 0)

    def loop_body(i, carry):
        for b in (0, 1):
            c = 2 * i + b

            @pl.when(c + 1 < _NCHUNK)
            def _():
                fire(c + 1, 1 - b)

            wait_gathers(c, b)

            @pl.when(c >= 2)
            def _():
                pltpu.make_async_copy(scbuf.at[b], out.at[pl.ds(base, _CB)],
                                      ssems[b]).wait()

            compute(c, b)
            pltpu.make_async_copy(scbuf.at[b],
                                  out.at[pl.ds(base + c * _CB, _CB)],
                                  ssems[b]).start()
        return carry

    lax.fori_loop(0, _NCHUNK // 2, loop_body, 0)

    pltpu.make_async_copy(scbuf.at[0], out.at[pl.ds(base, _CB)], ssem0).wait()
    pltpu.make_async_copy(scbuf.at[1], out.at[pl.ds(base, _CB)], ssem1).wait()


_score_call = pl.kernel(
    _score_body,
    out_type=jax.ShapeDtypeStruct((_TP,), jnp.float32),
    mesh=plsc.VectorSubcoreMesh(core_axis_name="c", subcore_axis_name="s",
                                num_cores=_NC, num_subcores=_NS),
    scratch_types=[
        pltpu.VMEM((_TPW,), jnp.int32),
        pltpu.VMEM((_TPW,), jnp.int32),
        pltpu.VMEM((_TPW,), jnp.int32),
        pltpu.VMEM((2, _CB, _D), jnp.float32),
        pltpu.VMEM((2, _CB, _D), jnp.float32),
        pltpu.VMEM((2, _CB, _D), jnp.float32),
        pltpu.VMEM((2, _CB), jnp.float32),
        pltpu.SemaphoreType.DMA,
        pltpu.SemaphoreType.DMA,
        pltpu.SemaphoreType.DMA,
        pltpu.SemaphoreType.DMA,
    ],
)


def _loss_body(emb_ref, wrel_ref, sc_ref, y_ref, mk_ref, out_ref, acc_ref):
    i = pl.program_id(0)

    @pl.when(i == 0)
    def _():
        acc_ref[0] = 0.0
        acc_ref[1] = 0.0
        acc_ref[2] = jnp.sum(wrel_ref[...] ** 2)

    acc_ref[0] += jnp.sum(emb_ref[...] ** 2)
    s = sc_ref[...]
    y = y_ref[...]
    m = mk_ref[...]
    bce = jnp.maximum(s, 0.0) - s * y + jnp.log1p(jnp.exp(-jnp.abs(s)))
    acc_ref[1] += jnp.sum(m * bce)

    @pl.when(i == _GB - 1)
    def _():
        out_ref[0, 0] = (acc_ref[1] / _T
                         + _REG * (acc_ref[0] / (_N * _D)
                                   + acc_ref[2] / (_R * _D)))


_loss_call = pl.pallas_call(
    _loss_body,
    out_shape=jax.ShapeDtypeStruct((1, 1), jnp.float32),
    grid=(_GB,),
    in_specs=[
        pl.BlockSpec((_EB, _D), lambda i: (i, 0)),
        pl.BlockSpec((_WPAD, _D), lambda i: (0, 0)),
        pl.BlockSpec((_SB, _D), lambda i: (i, 0)),
        pl.BlockSpec((_SB, _D), lambda i: (i, 0)),
        pl.BlockSpec((_SB, _D), lambda i: (i, 0)),
    ],
    out_specs=pl.BlockSpec(memory_space=pltpu.SMEM),
    scratch_shapes=[pltpu.SMEM((4,), jnp.float32)],
)


def kernel(embed, heads, rels, tails, labels, w_relation):
    zpad = jnp.zeros((_PAD,), jnp.int32)
    hp = jnp.concatenate([heads, zpad])
    rp = jnp.concatenate([rels, zpad])
    tp = jnp.concatenate([tails, zpad])
    scores = _score_call(embed, hp, rp, tp, w_relation)

    y2 = jnp.pad(labels.astype(jnp.float32), (0, _PAD)).reshape(_SROWS, _D)
    m2 = (jnp.arange(_TP, dtype=jnp.int32) < _T).astype(
        jnp.float32).reshape(_SROWS, _D)
    s2 = scores.reshape(_SROWS, _D)
    w512 = jnp.pad(w_relation, ((0, _WPAD - _R), (0, 0)))
    out = _loss_call(embed, w512, s2, y2, m2)
    return out[0, 0]


# speedup vs baseline: 2.1054x; 2.1054x over previous
import jax
import jax.numpy as jnp
from jax import lax
from jax.experimental import pallas as pl
from jax.experimental.pallas import tpu as pltpu
from jax.experimental.pallas import tpu_sc as plsc

_N, _D, _R, _T = 100000, 128, 500, 200000
_REG = 0.01
_NC, _NS = 2, 16
_NW = _NC * _NS
_CB = 128
_NCHUNK = 50
_TPW = _CB * _NCHUNK
_TP = _NW * _TPW
_PAD = _TP - _T
_NCH_TOT = _TP // _CB

_GB = 10
_EB = _N // _GB
_SROWS = _TP // _D
_SB = _SROWS // _GB
_WPAD = 512


def _score_body(embed, idx3_h, wrel, out,
                ibuf, sbuf, rbuf, obuf, scbuf,
                gsem0, gsem1, isem0, isem1, ssem0, ssem1):
    wid = lax.axis_index("s") * _NC + lax.axis_index("c")
    cbase = wid * _NCHUNK
    base = wid * _TPW

    gsems = (gsem0, gsem1)
    isems = (isem0, isem1)
    ssems = (ssem0, ssem1)

    def idx_copy(c, b):
        return pltpu.make_async_copy(idx3_h.at[cbase + c], ibuf.at[b],
                                     isems[b])

    def gather_descs(b):
        return (
            pltpu.make_async_copy(embed.at[ibuf.at[b, 0]], sbuf.at[b],
                                  gsems[b]),
            pltpu.make_async_copy(wrel.at[ibuf.at[b, 1]], rbuf.at[b],
                                  gsems[b]),
            pltpu.make_async_copy(embed.at[ibuf.at[b, 2]], obuf.at[b],
                                  gsems[b]),
        )

    def fire(b):
        for dsc in gather_descs(b):
            dsc.start()

    def wait_gathers(b):
        for dsc in gather_descs(b):
            dsc.wait()

    lane = lax.iota(jnp.int32, 16)

    def compute(c, b):
        def group(g, carry):
            def row(r, vec):
                t = g * 16 + r
                acc = jnp.zeros((16,), jnp.float32)
                for j in range(_D // 16):
                    sv = sbuf[b, t, pl.ds(j * 16, 16)]
                    rv = rbuf[b, t, pl.ds(j * 16, 16)]
                    ov = obuf[b, t, pl.ds(j * 16, 16)]
                    acc = acc + sv * rv * ov
                return jnp.where(lane == r, jnp.sum(acc), vec)

            vec = lax.fori_loop(0, 16, row, jnp.zeros((16,), jnp.float32),
                                unroll=4)
            scbuf[b, pl.ds(g * 16, 16)] = vec
            return carry

        lax.fori_loop(0, _CB // 16, group, 0)

    pltpu.sync_copy(idx3_h.at[cbase], ibuf.at[0])
    fire(0)
    idx_copy(1, 1).start()

    def loop_body(i, carry):
        for b in (0, 1):
            c = 2 * i + b

            @pl.when(c + 1 < _NCHUNK)
            def _():
                idx_copy(c + 1, 1 - b).wait()
                fire(1 - b)

            wait_gathers(b)

            @pl.when(c + 2 < _NCHUNK)
            def _():
                idx_copy(c + 2, b).start()

            @pl.when(c >= 2)
            def _():
                pltpu.make_async_copy(scbuf.at[b], out.at[pl.ds(base, _CB)],
                                      ssems[b]).wait()

            compute(c, b)
            pltpu.make_async_copy(scbuf.at[b],
                                  out.at[pl.ds(base + c * _CB, _CB)],
                                  ssems[b]).start()
        return carry

    lax.fori_loop(0, _NCHUNK // 2, loop_body, 0)

    pltpu.make_async_copy(scbuf.at[0], out.at[pl.ds(base, _CB)], ssem0).wait()
    pltpu.make_async_copy(scbuf.at[1], out.at[pl.ds(base, _CB)], ssem1).wait()


_score_call = pl.kernel(
    _score_body,
    out_type=jax.ShapeDtypeStruct((_TP,), jnp.float32),
    mesh=plsc.VectorSubcoreMesh(core_axis_name="c", subcore_axis_name="s",
                                num_cores=_NC, num_subcores=_NS),
    compiler_params=pltpu.CompilerParams(needs_layout_passes=False),
    scratch_types=[
        pltpu.VMEM((2, 3, _CB), jnp.int32),
        pltpu.VMEM((2, _CB, _D), jnp.float32),
        pltpu.VMEM((2, _CB, _D), jnp.float32),
        pltpu.VMEM((2, _CB, _D), jnp.float32),
        pltpu.VMEM((2, _CB), jnp.float32),
        pltpu.SemaphoreType.DMA,
        pltpu.SemaphoreType.DMA,
        pltpu.SemaphoreType.DMA,
        pltpu.SemaphoreType.DMA,
        pltpu.SemaphoreType.DMA,
        pltpu.SemaphoreType.DMA,
    ],
)


def _loss_body(emb_ref, wrel_ref, sc_ref, y_ref, mk_ref, out_ref, acc_ref):
    i = pl.program_id(0)

    @pl.when(i == 0)
    def _():
        acc_ref[0] = 0.0
        acc_ref[1] = 0.0
        acc_ref[2] = jnp.sum(wrel_ref[...] ** 2)

    acc_ref[0] += jnp.sum(emb_ref[...] ** 2)
    s = sc_ref[...]
    y = y_ref[...]
    m = mk_ref[...]
    bce = jnp.maximum(s, 0.0) - s * y + jnp.log1p(jnp.exp(-jnp.abs(s)))
    acc_ref[1] += jnp.sum(m * bce)

    @pl.when(i == _GB - 1)
    def _():
        out_ref[0, 0] = (acc_ref[1] / _T
                         + _REG * (acc_ref[0] / (_N * _D)
                                   + acc_ref[2] / (_R * _D)))


_loss_call = pl.pallas_call(
    _loss_body,
    out_shape=jax.ShapeDtypeStruct((1, 1), jnp.float32),
    grid=(_GB,),
    in_specs=[
        pl.BlockSpec((_EB, _D), lambda i: (i, 0)),
        pl.BlockSpec((_WPAD, _D), lambda i: (0, 0)),
        pl.BlockSpec((_SB, _D), lambda i: (i, 0)),
        pl.BlockSpec((_SB, _D), lambda i: (i, 0)),
        pl.BlockSpec((_SB, _D), lambda i: (i, 0)),
    ],
    out_specs=pl.BlockSpec(memory_space=pltpu.SMEM),
    scratch_shapes=[pltpu.SMEM((4,), jnp.float32)],
)


def kernel(embed, heads, rels, tails, labels, w_relation):
    zpad = jnp.zeros((_PAD,), jnp.int32)
    hp = jnp.concatenate([heads, zpad]).reshape(_NCH_TOT, _CB)
    rp = jnp.concatenate([rels, zpad]).reshape(_NCH_TOT, _CB)
    tp = jnp.concatenate([tails, zpad]).reshape(_NCH_TOT, _CB)
    idx3 = jnp.stack([hp, rp, tp], axis=1)
    scores = _score_call(embed, idx3, w_relation)

    y2 = jnp.pad(labels.astype(jnp.float32), (0, _PAD)).reshape(_SROWS, _D)
    m2 = (jnp.arange(_TP, dtype=jnp.int32) < _T).astype(
        jnp.float32).reshape(_SROWS, _D)
    s2 = scores.reshape(_SROWS, _D)
    w512 = jnp.pad(w_relation, ((0, _WPAD - _R), (0, 0)))
    out = _loss_call(embed, w512, s2, y2, m2)
    return out[0, 0]
